# baseline (speedup 1.0000x reference)
import functools

import jax
import jax.numpy as jnp
from jax import lax
from jax.experimental import pallas as pl
from jax.experimental.pallas import tpu as pltpu
from jax.experimental.pallas import tpu_sc as plsc

N = 256
P = 257
H = 768
D = P * H
NW = 32
CW = 6144
TAIL = D - NW * CW
TAIL_BASE = NW * CW
NV = CW // 16
NVT = TAIL // 16
ROWS = 9
UNROLL = 8
BPW = N // NW


def _scale_rows(cache, g, width):
    for r in range(ROWS):
        def body(jj, _, r=r):
            for u in range(UNROLL):
                sl = pl.ds(r * width + (jj * UNROLL + u) * 16, 16)
                cache[sl] = cache[sl] * g
            return 0
        lax.fori_loop(0, width // 16 // UNROLL, body, 0)


def _sc_kernel(x_hbm, ids_hbm, tbl_hbm, gate_hbm, out_hbm,
               idx_v, g_v, tcache, tail_cache, buf0, buf1):
    c = lax.axis_index("c")
    s = lax.axis_index("s")
    wid = c * 16 + s
    base = pl.multiple_of(wid * CW, 256)

    pltpu.sync_copy(ids_hbm, idx_v.at[pl.ds(0, N)])
    pltpu.sync_copy(gate_hbm, g_v)
    graw = g_v[...]
    g = 1.0 - 2.0 / (jnp.exp(2.0 * graw) + 1.0)

    for r in range(ROWS):
        pltpu.sync_copy(tbl_hbm.at[pl.ds(r * D + base, CW)],
                        tcache.at[pl.ds(r * CW, CW)])
        pltpu.sync_copy(tbl_hbm.at[pl.ds(r * D + TAIL_BASE, TAIL)],
                        tail_cache.at[pl.ds(r * TAIL, TAIL)])
    _scale_rows(tcache, g, CW)
    _scale_rows(tail_cache, g, TAIL)

    def step(i, _):
        rid = idx_v[pl.ds(i, 16)][0]
        tbase = pl.multiple_of(rid * CW, 256)
        pltpu.sync_copy(x_hbm.at[pl.ds(i * D + base, CW)], buf0)

        def add(jj, _):
            for u in range(UNROLL):
                off = (jj * UNROLL + u) * 16
                plsc.addupdate(buf0.at[pl.ds(off, 16)],
                               tcache[pl.ds(tbase + off, 16)])
            return 0
        lax.fori_loop(0, NV // UNROLL, add, 0)
        pltpu.sync_copy(buf0, out_hbm.at[pl.ds(i * D + base, CW)])
        return 0
    lax.fori_loop(0, N, step, 0)

    def tstep(k, _):
        i = wid * BPW + k
        rid = idx_v[pl.ds(i, 16)][0]
        tbase = pl.multiple_of(rid * TAIL, 256)
        pltpu.sync_copy(x_hbm.at[pl.ds(i * D + TAIL_BASE, TAIL)],
                        buf1.at[pl.ds(0, TAIL)])

        def add(jj, _):
            for u in range(UNROLL):
                off = (jj * UNROLL + u) * 16
                plsc.addupdate(buf1.at[pl.ds(off, 16)],
                               tail_cache[pl.ds(tbase + off, 16)])
            return 0
        lax.fori_loop(0, NVT // UNROLL, add, 0)
        pltpu.sync_copy(buf1.at[pl.ds(0, TAIL)],
                        out_hbm.at[pl.ds(i * D + TAIL_BASE, TAIL)])
        return 0
    lax.fori_loop(0, BPW, tstep, 0)


_mesh = plsc.VectorSubcoreMesh(core_axis_name="c", subcore_axis_name="s")

_call = functools.partial(
    pl.kernel,
    mesh=_mesh,
    out_type=jax.ShapeDtypeStruct((N * D,), jnp.float32),
    scratch_types=[
        pltpu.VMEM((N + 16,), jnp.int32),
        pltpu.VMEM((16,), jnp.float32),
        pltpu.VMEM((ROWS * CW,), jnp.float32),
        pltpu.VMEM((ROWS * TAIL,), jnp.float32),
        pltpu.VMEM((CW,), jnp.float32),
        pltpu.VMEM((CW,), jnp.float32),
    ],
)(_sc_kernel)


def kernel(x, aspect_ratio_ids, table, gate):
    x1 = x.reshape(N * D)
    ids = aspect_ratio_ids.astype(jnp.int32)
    tbl1 = table.reshape(ROWS * D)
    g16 = jnp.broadcast_to(gate.astype(jnp.float32).reshape(()), (16,))
    out = _call(x1, ids, tbl1, g16)
    return out.reshape(N, P, H)


# speedup vs baseline: 1.3167x; 1.3167x over previous
import functools

import jax
import jax.numpy as jnp
from jax import lax
from jax.experimental import pallas as pl
from jax.experimental.pallas import tpu as pltpu
from jax.experimental.pallas import tpu_sc as plsc

N = 256
P = 257
H = 768
D = P * H
NW = 32
CW = 6144
TAIL = D - NW * CW
TAIL_BASE = NW * CW
NV = CW // 16
NVT = TAIL // 16
ROWS = 9
NBUF = 4
UNROLL = 8
BPW = N // NW


def _scale_rows(cache, g, width):
    for r in range(ROWS):
        def body(jj, _, r=r):
            for u in range(UNROLL):
                sl = pl.ds(r * width + (jj * UNROLL + u) * 16, 16)
                cache[sl] = cache[sl] * g
            return 0
        lax.fori_loop(0, width // 16 // UNROLL, body, 0)


def _sc_kernel(x_hbm, ids_hbm, tbl_hbm, gate_hbm, out_hbm,
               idx_v, g_v, tcache, tail_cache,
               b0, b1, b2, b3, buf1,
               si0, si1, si2, si3, so0, so1, so2, so3):
    bufs = [b0, b1, b2, b3]
    sems_in = [si0, si1, si2, si3]
    sems_out = [so0, so1, so2, so3]
    c = lax.axis_index("c")
    s = lax.axis_index("s")
    wid = c * 16 + s
    base = pl.multiple_of(wid * CW, 256)

    pltpu.sync_copy(ids_hbm, idx_v.at[pl.ds(0, N)])
    pltpu.sync_copy(gate_hbm, g_v)
    graw = g_v[...]
    g = 1.0 - 2.0 / (jnp.exp(2.0 * graw) + 1.0)

    for r in range(ROWS):
        pltpu.sync_copy(tbl_hbm.at[pl.ds(r * D + base, CW)],
                        tcache.at[pl.ds(r * CW, CW)])
        pltpu.sync_copy(tbl_hbm.at[pl.ds(r * D + TAIL_BASE, TAIL)],
                        tail_cache.at[pl.ds(r * TAIL, TAIL)])
    _scale_rows(tcache, g, CW)
    _scale_rows(tail_cache, g, TAIL)

    def fill(j, slot):
        return pltpu.make_async_copy(
            x_hbm.at[pl.ds(j * D + base, CW)], bufs[slot], sems_in[slot])

    def drain(j, slot):
        return pltpu.make_async_copy(
            bufs[slot], out_hbm.at[pl.ds(j * D + base, CW)], sems_out[slot])

    fill(0, 0).start()
    fill(1, 1).start()

    def group(grp, _):
        for b in range(NBUF):
            i = grp * NBUF + b
            c = (b + 2) % NBUF

            @pl.when(i >= 2)
            def _():
                drain(i - 2, c).wait()

            @pl.when(i + 2 < N)
            def _():
                fill(i + 2, c).start()

            fill(i, b).wait()
            rid = idx_v[pl.ds(i, 16)][0]
            tbase = pl.multiple_of(rid * CW, 256)

            def add(jj, _, b=b, tbase=tbase):
                for u in range(UNROLL):
                    off = (jj * UNROLL + u) * 16
                    plsc.addupdate(bufs[b].at[pl.ds(off, 16)],
                                   tcache[pl.ds(tbase + off, 16)])
                return 0
            lax.fori_loop(0, NV // UNROLL, add, 0)
            drain(i, b).start()
        return 0
    lax.fori_loop(0, N // NBUF, group, 0)

    drain(N - 2, (N - 2) % NBUF).wait()
    drain(N - 1, (N - 1) % NBUF).wait()

    def tstep(k, _):
        i = wid * BPW + k
        rid = idx_v[pl.ds(i, 16)][0]
        tbase = pl.multiple_of(rid * TAIL, 256)
        pltpu.sync_copy(x_hbm.at[pl.ds(i * D + TAIL_BASE, TAIL)],
                        buf1.at[pl.ds(0, TAIL)])

        def add(jj, _):
            for u in range(UNROLL):
                off = (jj * UNROLL + u) * 16
                plsc.addupdate(buf1.at[pl.ds(off, 16)],
                               tail_cache[pl.ds(tbase + off, 16)])
            return 0
        lax.fori_loop(0, NVT // UNROLL, add, 0)
        pltpu.sync_copy(buf1.at[pl.ds(0, TAIL)],
                        out_hbm.at[pl.ds(i * D + TAIL_BASE, TAIL)])
        return 0
    lax.fori_loop(0, BPW, tstep, 0)


_mesh = plsc.VectorSubcoreMesh(core_axis_name="c", subcore_axis_name="s")

_call = functools.partial(
    pl.kernel,
    mesh=_mesh,
    out_type=jax.ShapeDtypeStruct((N * D,), jnp.float32),
    scratch_types=[
        pltpu.VMEM((N + 16,), jnp.int32),
        pltpu.VMEM((16,), jnp.float32),
        pltpu.VMEM((ROWS * CW,), jnp.float32),
        pltpu.VMEM((ROWS * TAIL,), jnp.float32),
        pltpu.VMEM((CW,), jnp.float32),
        pltpu.VMEM((CW,), jnp.float32),
        pltpu.VMEM((CW,), jnp.float32),
        pltpu.VMEM((CW,), jnp.float32),
        pltpu.VMEM((CW,), jnp.float32),
        pltpu.SemaphoreType.DMA,
        pltpu.SemaphoreType.DMA,
        pltpu.SemaphoreType.DMA,
        pltpu.SemaphoreType.DMA,
        pltpu.SemaphoreType.DMA,
        pltpu.SemaphoreType.DMA,
        pltpu.SemaphoreType.DMA,
        pltpu.SemaphoreType.DMA,
    ],
)(_sc_kernel)


def kernel(x, aspect_ratio_ids, table, gate):
    x1 = x.reshape(N * D)
    ids = aspect_ratio_ids.astype(jnp.int32)
    tbl1 = table.reshape(ROWS * D)
    g16 = jnp.broadcast_to(gate.astype(jnp.float32).reshape(()), (16,))
    out = _call(x1, ids, tbl1, g16)
    return out.reshape(N, P, H)


# speedup vs baseline: 1.7626x; 1.3387x over previous
import functools

import jax
import jax.numpy as jnp
from jax import lax
from jax.experimental import pallas as pl
from jax.experimental.pallas import tpu as pltpu
from jax.experimental.pallas import tpu_sc as plsc

N = 256
P = 257
H = 768
D = P * H
NW = 32
CW = 6144
TAIL = D - NW * CW
TAIL_BASE = NW * CW
NV = CW // 16
NVT = TAIL // 16
ROWS = 9
NBUF = 4
UNROLL = 8
BPW = N // NW


def _scale_rows(cache, g, width):
    @plsc.parallel_loop(0, ROWS * width, step=16, unroll=UNROLL)
    def _(off):
        sl = pl.ds(off, 16)
        cache[sl] = cache[sl] * g


def _sc_kernel(x_hbm, ids_hbm, tbl_hbm, gate_hbm, out_hbm,
               idx_v, g_v, tcache, tail_cache,
               b0, b1, b2, b3, buf1,
               si0, si1, si2, si3, so0, so1, so2, so3):
    bufs = [b0, b1, b2, b3]
    sems_in = [si0, si1, si2, si3]
    sems_out = [so0, so1, so2, so3]
    c = lax.axis_index("c")
    s = lax.axis_index("s")
    wid = c * 16 + s
    base = pl.multiple_of(wid * CW, 256)

    pltpu.sync_copy(ids_hbm, idx_v.at[pl.ds(0, N)])
    pltpu.sync_copy(gate_hbm, g_v)
    graw = g_v[...]
    g = 1.0 - 2.0 / (jnp.exp(2.0 * graw) + 1.0)

    for r in range(ROWS):
        pltpu.sync_copy(tbl_hbm.at[pl.ds(r * D + base, CW)],
                        tcache.at[pl.ds(r * CW, CW)])
        pltpu.sync_copy(tbl_hbm.at[pl.ds(r * D + TAIL_BASE, TAIL)],
                        tail_cache.at[pl.ds(r * TAIL, TAIL)])
    _scale_rows(tcache, g, CW)
    _scale_rows(tail_cache, g, TAIL)

    def fill(j, slot):
        return pltpu.make_async_copy(
            x_hbm.at[pl.ds(j * D + base, CW)], bufs[slot], sems_in[slot])

    def drain(j, slot):
        return pltpu.make_async_copy(
            bufs[slot], out_hbm.at[pl.ds(j * D + base, CW)], sems_out[slot])

    fill(0, 0).start()
    fill(1, 1).start()

    def group(grp, _):
        for b in range(NBUF):
            i = grp * NBUF + b
            c = (b + 2) % NBUF

            @pl.when(i >= 2)
            def _():
                drain(i - 2, c).wait()

            @pl.when(i + 2 < N)
            def _():
                fill(i + 2, c).start()

            fill(i, b).wait()
            rid = idx_v[pl.ds(i, 16)][0]
            tbase = pl.multiple_of(rid * CW, 256)

            buf = bufs[b]

            @plsc.parallel_loop(0, CW, step=16, unroll=UNROLL)
            def _(off):
                plsc.addupdate(buf.at[pl.ds(off, 16)],
                               tcache[pl.ds(tbase + off, 16)])
            drain(i, b).start()
        return 0
    lax.fori_loop(0, N // NBUF, group, 0)

    drain(N - 2, (N - 2) % NBUF).wait()
    drain(N - 1, (N - 1) % NBUF).wait()

    def tstep(k, _):
        i = wid * BPW + k
        rid = idx_v[pl.ds(i, 16)][0]
        tbase = pl.multiple_of(rid * TAIL, 256)
        pltpu.sync_copy(x_hbm.at[pl.ds(i * D + TAIL_BASE, TAIL)],
                        buf1.at[pl.ds(0, TAIL)])

        @plsc.parallel_loop(0, TAIL, step=16, unroll=UNROLL)
        def _(off):
            plsc.addupdate(buf1.at[pl.ds(off, 16)],
                           tail_cache[pl.ds(tbase + off, 16)])
        pltpu.sync_copy(buf1.at[pl.ds(0, TAIL)],
                        out_hbm.at[pl.ds(i * D + TAIL_BASE, TAIL)])
        return 0
    lax.fori_loop(0, BPW, tstep, 0)


_mesh = plsc.VectorSubcoreMesh(core_axis_name="c", subcore_axis_name="s")

_call = functools.partial(
    pl.kernel,
    mesh=_mesh,
    out_type=jax.ShapeDtypeStruct((N * D,), jnp.float32),
    scratch_types=[
        pltpu.VMEM((N + 16,), jnp.int32),
        pltpu.VMEM((16,), jnp.float32),
        pltpu.VMEM((ROWS * CW,), jnp.float32),
        pltpu.VMEM((ROWS * TAIL,), jnp.float32),
        pltpu.VMEM((CW,), jnp.float32),
        pltpu.VMEM((CW,), jnp.float32),
        pltpu.VMEM((CW,), jnp.float32),
        pltpu.VMEM((CW,), jnp.float32),
        pltpu.VMEM((CW,), jnp.float32),
        pltpu.SemaphoreType.DMA,
        pltpu.SemaphoreType.DMA,
        pltpu.SemaphoreType.DMA,
        pltpu.SemaphoreType.DMA,
        pltpu.SemaphoreType.DMA,
        pltpu.SemaphoreType.DMA,
        pltpu.SemaphoreType.DMA,
        pltpu.SemaphoreType.DMA,
    ],
)(_sc_kernel)


def kernel(x, aspect_ratio_ids, table, gate):
    x1 = x.reshape(N * D)
    ids = aspect_ratio_ids.astype(jnp.int32)
    tbl1 = table.reshape(ROWS * D)
    g16 = jnp.broadcast_to(gate.astype(jnp.float32).reshape(()), (16,))
    out = _call(x1, ids, tbl1, g16)
    return out.reshape(N, P, H)


# speedup vs baseline: 2.6265x; 1.4902x over previous
import functools

import jax
import jax.numpy as jnp
from jax import lax
from jax.experimental import pallas as pl
from jax.experimental.pallas import tpu as pltpu
from jax.experimental.pallas import tpu_sc as plsc

N = 256
P = 257
H = 768
NW = 32
PT = (P + 7) // 8
ROWS = 9
NBUF = 4
BPW = N // NW


def _sc_kernel(x_hbm, ids_hbm, tbl_hbm, gate_hbm, out_hbm,
               idx_v, g_v, tcache, tailc,
               b0, b1, b2, b3, tbuf,
               si0, si1, si2, si3, so0, so1, so2, so3):
    bufs = [b0, b1, b2, b3]
    sems_in = [si0, si1, si2, si3]
    sems_out = [so0, so1, so2, so3]
    c = lax.axis_index("c")
    s = lax.axis_index("s")
    wid = c * 16 + s
    p0 = pl.multiple_of(wid * 8, 8)

    pltpu.sync_copy(ids_hbm, idx_v.at[pl.ds(0, N)])
    pltpu.sync_copy(gate_hbm, g_v)
    graw = g_v[...]
    g = 1.0 - 2.0 / (jnp.exp(2.0 * graw) + 1.0)

    for v in range(ROWS):
        pltpu.sync_copy(tbl_hbm.at[v, wid], tcache.at[pl.ds(v * 8, 8), :])
        pltpu.sync_copy(tbl_hbm.at[v, PT - 1, pl.ds(0, 1), :],
                        tailc.at[pl.ds(v, 1), :])

    @plsc.parallel_loop(0, H, step=16, unroll=2)
    def _(cc):
        sl = pl.ds(cc, 16)
        for r in range(ROWS * 8):
            tcache[r, sl] = tcache[r, sl] * g
        for v in range(ROWS):
            tailc[v, sl] = tailc[v, sl] * g

    def fill(j, slot):
        return pltpu.make_async_copy(
            x_hbm.at[j, pl.ds(p0, 8), :], bufs[slot], sems_in[slot])

    def drain(j, slot):
        return pltpu.make_async_copy(
            bufs[slot], out_hbm.at[j, pl.ds(p0, 8), :], sems_out[slot])

    fill(0, 0).start()
    fill(1, 1).start()

    def group(grp, _):
        for b in range(NBUF):
            i = grp * NBUF + b
            nxt = (b + 2) % NBUF

            @pl.when(i >= 2)
            def _():
                drain(i - 2, nxt).wait()

            @pl.when(i + 2 < N)
            def _():
                fill(i + 2, nxt).start()

            fill(i, b).wait()
            rid = idx_v[pl.ds(i, 16)][0]
            r0 = pl.multiple_of(rid * 8, 8)
            buf = bufs[b]

            @plsc.parallel_loop(0, H, step=16, unroll=2)
            def _(cc):
                sl = pl.ds(cc, 16)
                for r in range(8):
                    plsc.addupdate(buf.at[r, sl], tcache[r0 + r, sl])
            drain(i, b).start()
        return 0
    lax.fori_loop(0, N // NBUF, group, 0)

    drain(N - 2, (N - 2) % NBUF).wait()
    drain(N - 1, (N - 1) % NBUF).wait()

    def tstep(k, _):
        i = wid * BPW + k
        rid = idx_v[pl.ds(i, 16)][0]
        pltpu.sync_copy(x_hbm.at[i, pl.ds(P - 1, 1), :],
                        tbuf.at[pl.ds(0, 1), :])

        @plsc.parallel_loop(0, H, step=16, unroll=2)
        def _(cc):
            sl = pl.ds(cc, 16)
            plsc.addupdate(tbuf.at[0, sl], tailc[rid, sl])

        pltpu.sync_copy(tbuf.at[pl.ds(0, 1), :],
                        out_hbm.at[i, pl.ds(P - 1, 1), :])
        return 0
    lax.fori_loop(0, BPW, tstep, 0)


_mesh = plsc.VectorSubcoreMesh(core_axis_name="c", subcore_axis_name="s")

_call = functools.partial(
    pl.kernel,
    mesh=_mesh,
    out_type=jax.ShapeDtypeStruct((N, P, H), jnp.float32),
    compiler_params=pltpu.CompilerParams(use_tc_tiling_on_sc=True),
    scratch_types=[
        pltpu.VMEM((N + 16,), jnp.int32),
        pltpu.VMEM((16,), jnp.float32),
        pltpu.VMEM((ROWS * 8, H), jnp.float32),
        pltpu.VMEM((ROWS, H), jnp.float32),
        pltpu.VMEM((8, H), jnp.float32),
        pltpu.VMEM((8, H), jnp.float32),
        pltpu.VMEM((8, H), jnp.float32),
        pltpu.VMEM((8, H), jnp.float32),
        pltpu.VMEM((8, H), jnp.float32),
        pltpu.SemaphoreType.DMA,
        pltpu.SemaphoreType.DMA,
        pltpu.SemaphoreType.DMA,
        pltpu.SemaphoreType.DMA,
        pltpu.SemaphoreType.DMA,
        pltpu.SemaphoreType.DMA,
        pltpu.SemaphoreType.DMA,
        pltpu.SemaphoreType.DMA,
    ],
)(_sc_kernel)


def kernel(x, aspect_ratio_ids, table, gate):
    ids = aspect_ratio_ids.astype(jnp.int32)
    tbl3 = table.reshape(ROWS, P, H)
    tblp = jnp.pad(tbl3, ((0, 0), (0, PT * 8 - P), (0, 0)))
    tblp = tblp.reshape(ROWS, PT, 8, H)
    g16 = jnp.broadcast_to(gate.astype(jnp.float32).reshape(()), (16,))
    return _call(x, ids, tblp, g16)


# speedup vs baseline: 5.4998x; 2.0939x over previous
import functools

import jax
import jax.numpy as jnp
from jax import lax
from jax.experimental import pallas as pl
from jax.experimental.pallas import tpu as pltpu
from jax.experimental.pallas import tpu_sc as plsc

N = 256
P = 257
H = 768
NW = 32
NB = N // 8
ROWS = 9
PS = 9
NBUF = 4
NCH = (P - 1) * NB // NW


def _sc_kernel(x_hbm, ids_hbm, tbl_hbm, gate_hbm, out_hbm,
               idx_v, g_v, tcache,
               b0, b1, b2, b3,
               si0, si1, si2, si3, so0, so1, so2, so3):
    bufs = [b0, b1, b2, b3]
    sems_in = [si0, si1, si2, si3]
    sems_out = [so0, so1, so2, so3]
    c = lax.axis_index("c")
    s = lax.axis_index("s")
    wid = c * 16 + s

    pltpu.sync_copy(ids_hbm, idx_v.at[pl.ds(0, N)])
    pltpu.sync_copy(gate_hbm, g_v)
    graw = g_v[...]
    g = 1.0 - 2.0 / (jnp.exp(2.0 * graw) + 1.0)

    pltpu.sync_copy(tbl_hbm.at[wid], tcache)

    @plsc.parallel_loop(0, H, step=16, unroll=2)
    def _(cc):
        sl = pl.ds(cc, 16)
        for r in range(ROWS * PS):
            tcache[r, sl] = tcache[r, sl] * g

    def chunk_coords(t):
        return wid * 8 + lax.shift_right_logical(t, 5), lax.bitwise_and(t, 31)

    def fill(t, slot):
        p, bg = chunk_coords(t)
        return pltpu.make_async_copy(
            x_hbm.at[p, pl.ds(bg * 8, 8), :], bufs[slot], sems_in[slot])

    def drain(t, slot):
        p, bg = chunk_coords(t)
        return pltpu.make_async_copy(
            bufs[slot], out_hbm.at[p, pl.ds(bg * 8, 8), :], sems_out[slot])

    def add_rows(buf, ids16, pslot):
        @plsc.parallel_loop(0, H, step=16, unroll=2)
        def _(cc):
            sl = pl.ds(cc, 16)
            for r in range(8):
                row = ids16[r] * PS + pslot
                plsc.addupdate(buf.at[r, sl], tcache[row, sl])

    fill(0, 0).start()
    fill(1, 1).start()

    def group(grp, _):
        for b in range(NBUF):
            t = grp * NBUF + b
            nxt = (b + 2) % NBUF

            @pl.when(t >= 2)
            def _():
                drain(t - 2, nxt).wait()

            @pl.when(t + 2 < NCH)
            def _():
                fill(t + 2, nxt).start()

            fill(t, b).wait()
            pslot = lax.shift_right_logical(t, 5)
            bg = lax.bitwise_and(t, 31)
            ids16 = idx_v[pl.ds(bg * 8, 16)]
            add_rows(bufs[b], ids16, pslot)
            drain(t, b).start()
        return 0
    lax.fori_loop(0, NCH // NBUF, group, 0)

    drain(NCH - 2, (NCH - 2) % NBUF).wait()
    drain(NCH - 1, (NCH - 1) % NBUF).wait()

    pltpu.sync_copy(x_hbm.at[P - 1, pl.ds(wid * 8, 8), :], b0)
    ids16 = idx_v[pl.ds(wid * 8, 16)]
    add_rows(b0, ids16, 8)
    pltpu.sync_copy(b0, out_hbm.at[P - 1, pl.ds(wid * 8, 8), :])


_mesh = plsc.VectorSubcoreMesh(core_axis_name="c", subcore_axis_name="s")

_call = functools.partial(
    pl.kernel,
    mesh=_mesh,
    out_type=jax.ShapeDtypeStruct((P, N, H), jnp.float32),
    compiler_params=pltpu.CompilerParams(use_tc_tiling_on_sc=True),
    scratch_types=[
        pltpu.VMEM((N + 16,), jnp.int32),
        pltpu.VMEM((16,), jnp.float32),
        pltpu.VMEM((ROWS * PS, H), jnp.float32),
        pltpu.VMEM((8, H), jnp.float32),
        pltpu.VMEM((8, H), jnp.float32),
        pltpu.VMEM((8, H), jnp.float32),
        pltpu.VMEM((8, H), jnp.float32),
        pltpu.SemaphoreType.DMA,
        pltpu.SemaphoreType.DMA,
        pltpu.SemaphoreType.DMA,
        pltpu.SemaphoreType.DMA,
        pltpu.SemaphoreType.DMA,
        pltpu.SemaphoreType.DMA,
        pltpu.SemaphoreType.DMA,
        pltpu.SemaphoreType.DMA,
    ],
)(_sc_kernel)


def kernel(x, aspect_ratio_ids, table, gate):
    ids = aspect_ratio_ids.astype(jnp.int32)
    xt = jnp.transpose(x, (1, 0, 2))
    tbl3 = table.reshape(ROWS, P, H)
    main = tbl3[:, :P - 1, :].reshape(ROWS, NW, 8, H)
    tailp = jnp.broadcast_to(tbl3[:, None, P - 1:, :], (ROWS, NW, 1, H))
    tblw = jnp.concatenate([main, tailp], axis=2)
    tblw = tblw.transpose(1, 0, 2, 3).reshape(NW, ROWS * PS, H)
    g16 = jnp.broadcast_to(gate.astype(jnp.float32).reshape(()), (16,))
    out = _call(xt, ids, tblw, g16)
    return jnp.transpose(out, (1, 0, 2))


# speedup vs baseline: 6.6680x; 1.2124x over previous
import functools

import jax
import jax.numpy as jnp
from jax import lax
from jax.experimental import pallas as pl
from jax.experimental.pallas import tpu as pltpu
from jax.experimental.pallas import tpu_sc as plsc

N = 256
P = 257
H = 768
NW = 32
CB = 16
NG = N // CB
ROWS = 9
NBUF = 4
NCH = (P - 1) * NG // NW


def _sc_kernel(x_hbm, ids_hbm, tbl_hbm, tail_hbm, gate_hbm, out_hbm,
               idx_v, g_v, tcache, tailc,
               b0, b1, b2, b3,
               si0, si1, si2, si3, so0, so1, so2, so3):
    bufs = [b0, b1, b2, b3]
    sems_in = [si0, si1, si2, si3]
    sems_out = [so0, so1, so2, so3]
    c = lax.axis_index("c")
    s = lax.axis_index("s")
    wid = c * 16 + s

    pltpu.sync_copy(ids_hbm, idx_v.at[pl.ds(0, N)])
    pltpu.sync_copy(gate_hbm, g_v)
    graw = g_v[...]
    g = 1.0 - 2.0 / (jnp.exp(2.0 * graw) + 1.0)

    for v in range(ROWS):
        pltpu.sync_copy(tbl_hbm.at[v, wid], tcache.at[pl.ds(v * 8, 8), :])
        pltpu.sync_copy(tail_hbm.at[v], tailc.at[pl.ds(v, 1), :])

    @plsc.parallel_loop(0, H, step=16, unroll=2)
    def _(cc):
        sl = pl.ds(cc, 16)
        for r in range(ROWS * 8):
            tcache[r, sl] = tcache[r, sl] * g
        for r in range(ROWS):
            tailc[r, sl] = tailc[r, sl] * g

    def fill(t, slot):
        p = wid * 8 + lax.shift_right_logical(t, 4)
        bg = lax.bitwise_and(t, 15)
        return pltpu.make_async_copy(
            x_hbm.at[p, pl.ds(bg * CB, CB), :], bufs[slot], sems_in[slot])

    def drain(t, slot):
        p = wid * 8 + lax.shift_right_logical(t, 4)
        bg = lax.bitwise_and(t, 15)
        return pltpu.make_async_copy(
            bufs[slot], out_hbm.at[p, pl.ds(bg * CB, CB), :], sems_out[slot])

    fill(0, 0).start()
    fill(1, 1).start()

    def group(grp, _):
        for b in range(NBUF):
            t = grp * NBUF + b
            nxt = (b + 2) % NBUF

            @pl.when(t >= 2)
            def _():
                drain(t - 2, nxt).wait()

            @pl.when(t + 2 < NCH)
            def _():
                fill(t + 2, nxt).start()

            fill(t, b).wait()
            pslot = lax.shift_right_logical(t, 4)
            bg = lax.bitwise_and(t, 15)
            ids16 = idx_v[pl.ds(bg * CB, 16)]
            buf = bufs[b]

            @plsc.parallel_loop(0, H, step=16, unroll=2)
            def _(cc):
                sl = pl.ds(cc, 16)
                for r in range(CB):
                    plsc.addupdate(buf.at[r, sl],
                                   tcache[ids16[r] * 8 + pslot, sl])
            drain(t, b).start()
        return 0
    lax.fori_loop(0, NCH // NBUF, group, 0)

    drain(NCH - 2, (NCH - 2) % NBUF).wait()
    drain(NCH - 1, (NCH - 1) % NBUF).wait()

    @pl.when(wid < NG)
    def _():
        pltpu.sync_copy(x_hbm.at[P - 1, pl.ds(wid * CB, CB), :], b0)
        ids16 = idx_v[pl.ds(wid * CB, 16)]

        @plsc.parallel_loop(0, H, step=16, unroll=2)
        def _(cc):
            sl = pl.ds(cc, 16)
            for r in range(CB):
                plsc.addupdate(b0.at[r, sl], tailc[ids16[r], sl])

        pltpu.sync_copy(b0, out_hbm.at[P - 1, pl.ds(wid * CB, CB), :])


_mesh = plsc.VectorSubcoreMesh(core_axis_name="c", subcore_axis_name="s")

_call = functools.partial(
    pl.kernel,
    mesh=_mesh,
    out_type=jax.ShapeDtypeStruct((P, N, H), jnp.float32),
    compiler_params=pltpu.CompilerParams(use_tc_tiling_on_sc=True),
    scratch_types=[
        pltpu.VMEM((N + 16,), jnp.int32),
        pltpu.VMEM((16,), jnp.float32),
        pltpu.VMEM((ROWS * 8, H), jnp.float32),
        pltpu.VMEM((ROWS, H), jnp.float32),
        pltpu.VMEM((CB, H), jnp.float32),
        pltpu.VMEM((CB, H), jnp.float32),
        pltpu.VMEM((CB, H), jnp.float32),
        pltpu.VMEM((CB, H), jnp.float32),
        pltpu.SemaphoreType.DMA,
        pltpu.SemaphoreType.DMA,
        pltpu.SemaphoreType.DMA,
        pltpu.SemaphoreType.DMA,
        pltpu.SemaphoreType.DMA,
        pltpu.SemaphoreType.DMA,
        pltpu.SemaphoreType.DMA,
        pltpu.SemaphoreType.DMA,
    ],
)(_sc_kernel)


def kernel(x, aspect_ratio_ids, table, gate):
    ids = aspect_ratio_ids.astype(jnp.int32)
    xt = jnp.transpose(x, (1, 0, 2))
    tbl3 = table.reshape(ROWS, P, H)
    tblp = tbl3[:, :P - 1, :].reshape(ROWS, NW, 8, H)
    tail = tbl3[:, P - 1:, :]
    g16 = jnp.broadcast_to(gate.astype(jnp.float32).reshape(()), (16,))
    out = _call(xt, ids, tblp, tail, g16)
    return jnp.transpose(out, (1, 0, 2))
